# baseline (speedup 1.0000x reference)
import functools

import jax
import jax.numpy as jnp
from jax import lax
from jax.experimental import pallas as pl
from jax.experimental.pallas import tpu as pltpu
from jax.experimental.pallas import tpu_sc as plsc

XW = 335
OW = 408
MAXL = 16
ALPHA = 20
P = 34
S15 = 15
F = 9
O = 12
NC = 2
NS = 16
NW = NC * NS
L = 16


def _sc_conv(xf, tabf, fltf, B):
    spw = B // NW
    ng = spw // L

    mesh = plsc.VectorSubcoreMesh(core_axis_name="c", subcore_axis_name="s")

    @functools.partial(
        pl.kernel,
        mesh=mesh,
        out_type=jax.ShapeDtypeStruct((B * OW,), jnp.float32),
        scratch_types=[
            pltpu.VMEM((L * XW,), jnp.float32),
            pltpu.VMEM((L * XW,), jnp.float32),
            pltpu.VMEM((spw * OW,), jnp.float32),
            pltpu.VMEM((MAXL * P,), jnp.int32),
            pltpu.VMEM((ALPHA * F,), jnp.float32),
            pltpu.SemaphoreType.DMA,
            pltpu.SemaphoreType.DMA,
            pltpu.SemaphoreType.DMA,
        ],
        compiler_params=pltpu.CompilerParams(needs_layout_passes=False),
    )
    def k(x_hbm, tab_hbm, flt_hbm, out_hbm, xa, xb, outs, tabs, flts,
          sa, sb, so):
        wid = lax.axis_index("s") * NC + lax.axis_index("c")
        base = wid * spw

        xbufs = (xa, xb)
        sems = (sa, sb)
        in_desc = [None, None]

        def start_in(g):
            in_desc[g % 2] = pltpu.async_copy(
                x_hbm.at[pl.ds((base + g * L) * XW, L * XW)],
                xbufs[g % 2], sems[g % 2])

        start_in(0)
        pltpu.sync_copy(tab_hbm, tabs)
        pltpu.sync_copy(flt_hbm, flts)

        lanes = lax.broadcasted_iota(jnp.int32, (L,), 0)
        rb = lanes * XW
        out_descs = []

        for g in range(ng):
            xs = xbufs[g % 2]
            in_desc[g % 2].wait()
            if g + 1 < ng:
                start_in(g + 1)

            ob = (lanes + g * L) * OW
            len_i = plsc.load_gather(xs, [rb]).astype(jnp.int32)
            tb = len_i * P

            def pbody(p, _, xs=xs, tb=tb, ob=ob):
                pv = plsc.load_gather(tabs, [tb + p])
                a = plsc.load_gather(xs, [rb + (301 + p)]).astype(jnp.int32) * F
                kf = [plsc.load_gather(flts, [a + f]) for f in range(F)]
                c0 = rb + lax.shift_right_logical(pv, 10) - 19
                c1 = rb + (pv & 1023) - 19
                s = [plsc.load_gather(xs, [c0 + c]) + plsc.load_gather(xs, [c1 + c])
                     for c in range(20)]
                po = ob + p * O
                for o in range(O):
                    acc = s[o] * kf[0]
                    for f in range(1, F):
                        acc = acc + s[o + f] * kf[f]
                    plsc.store_scatter(outs, [po + o], acc)
                return 0

            lax.fori_loop(0, P, pbody, 0, unroll=2)
            out_descs.append(pltpu.async_copy(
                outs.at[pl.ds(g * L * OW, L * OW)],
                out_hbm.at[pl.ds((base + g * L) * OW, L * OW)], so))

        for d in out_descs:
            d.wait()

    return k(xf, tabf, fltf)


def kernel(x, kernel, pocket_table):
    B = x.shape[0]
    t = pocket_table.astype(jnp.int32)
    packed = ((t[:, :, 0] * ALPHA) << 10) | (t[:, :, 1] * ALPHA)
    out = _sc_conv(
        x.reshape(-1),
        packed.reshape(-1),
        kernel.reshape(-1),
        B,
    )
    return out.reshape(B, OW)


# speedup vs baseline: 1.0261x; 1.0261x over previous
import functools

import jax
import jax.numpy as jnp
from jax import lax
from jax.experimental import pallas as pl
from jax.experimental.pallas import tpu as pltpu
from jax.experimental.pallas import tpu_sc as plsc

XW = 335
OW = 408
MAXL = 16
ALPHA = 20
P = 34
S15 = 15
F = 9
O = 12
NC = 2
NS = 16
NW = NC * NS
L = 16


def _sc_conv(xf, tabf, fltf, B):
    spw = B // NW
    ng = spw // L

    mesh = plsc.VectorSubcoreMesh(core_axis_name="c", subcore_axis_name="s")

    @functools.partial(
        pl.kernel,
        mesh=mesh,
        out_type=jax.ShapeDtypeStruct((B * OW,), jnp.float32),
        scratch_types=[
            pltpu.VMEM((L * XW,), jnp.float32),
            pltpu.VMEM((L * XW,), jnp.float32),
            pltpu.VMEM((spw * OW,), jnp.float32),
            pltpu.VMEM((MAXL * P,), jnp.int32),
            pltpu.VMEM((ALPHA * F,), jnp.float32),
            pltpu.SemaphoreType.DMA,
            pltpu.SemaphoreType.DMA,
            pltpu.SemaphoreType.DMA,
        ],
        compiler_params=pltpu.CompilerParams(needs_layout_passes=False),
    )
    def k(x_hbm, tab_hbm, flt_hbm, out_hbm, xa, xb, outs, tabs, flts,
          sa, sb, so):
        wid = lax.axis_index("s") * NC + lax.axis_index("c")
        base = wid * spw

        xbufs = (xa, xb)
        sems = (sa, sb)
        in_desc = [None, None]

        def start_in(g):
            in_desc[g % 2] = pltpu.async_copy(
                x_hbm.at[pl.ds((base + g * L) * XW, L * XW)],
                xbufs[g % 2], sems[g % 2])

        start_in(0)
        pltpu.sync_copy(tab_hbm, tabs)
        pltpu.sync_copy(flt_hbm, flts)

        lanes = lax.broadcasted_iota(jnp.int32, (L,), 0)
        rb = lanes * XW
        out_descs = []

        for g in range(ng):
            xs = xbufs[g % 2]
            in_desc[g % 2].wait()
            if g + 1 < ng:
                start_in(g + 1)

            ob = (lanes + g * L) * OW
            len_i = plsc.load_gather(xs, [rb]).astype(jnp.int32)
            tb = len_i * P

            def pbody(p, _, xs=xs, tb=tb, ob=ob):
                pv = plsc.load_gather(tabs, [tb + p])
                a = plsc.load_gather(xs, [rb + (301 + p)]).astype(jnp.int32) * F
                kf = [plsc.load_gather(flts, [a + f]) for f in range(F)]
                c0 = rb + lax.shift_right_logical(pv, 10) - 19
                c1 = rb + (pv & 1023) - 19
                s = [plsc.load_gather(xs, [c0 + c]) + plsc.load_gather(xs, [c1 + c])
                     for c in range(20)]
                po = ob + p * O
                for o in range(O):
                    acc = s[o] * kf[0]
                    for f in range(1, F):
                        acc = acc + s[o + f] * kf[f]
                    plsc.store_scatter(outs, [po + o], acc)
                return 0

            lax.fori_loop(0, P, pbody, 0, unroll=1)
            out_descs.append(pltpu.async_copy(
                outs.at[pl.ds(g * L * OW, L * OW)],
                out_hbm.at[pl.ds((base + g * L) * OW, L * OW)], so))

        for d in out_descs:
            d.wait()

    return k(xf, tabf, fltf)


def kernel(x, kernel, pocket_table):
    B = x.shape[0]
    t = pocket_table.astype(jnp.int32)
    packed = ((t[:, :, 0] * ALPHA) << 10) | (t[:, :, 1] * ALPHA)
    out = _sc_conv(
        x.reshape(-1),
        packed.reshape(-1),
        kernel.reshape(-1),
        B,
    )
    return out.reshape(B, OW)
